# baseline (speedup 1.0000x reference)
import functools

import jax
import jax.numpy as jnp
from jax import lax
from jax.experimental import pallas as pl
from jax.experimental.pallas import tpu as pltpu
from jax.experimental.pallas import tpu_sc as plsc

N_FACTORS = 32
BATCH = 16384
NC = 2
NS = 16
NW = NC * NS
BPW = BATCH // NW
CHUNK = 128
NCHUNK = BPW // CHUNK


def _mf_body(uidx_hbm, iidx_hbm, uf_hbm, if_hbm, out_hbm,
             uidx_v, iidx_v, u_v, i_v, out_v, sem):
    wid = lax.axis_index("s") * NC + lax.axis_index("c")
    pltpu.sync_copy(uidx_hbm.at[wid], uidx_v)
    pltpu.sync_copy(iidx_hbm.at[wid], iidx_v)
    copies = []
    for j in range(NCHUNK):
        dst = pl.ds(j * CHUNK, CHUNK)
        copies.append(pltpu.async_copy(uf_hbm.at[uidx_v.at[j]], u_v.at[dst], sem))
        copies.append(pltpu.async_copy(if_hbm.at[iidx_v.at[j]], i_v.at[dst], sem))
    for cp in copies:
        cp.wait()

    def group(g, carry):
        rows = g * 16 + lax.broadcasted_iota(jnp.int32, (16,), 0)
        acc = jnp.zeros((16,), jnp.float32)
        for d in range(N_FACTORS):
            cols = jnp.full((16,), d, jnp.int32)
            acc = acc + (plsc.load_gather(u_v, [rows, cols]) *
                         plsc.load_gather(i_v, [rows, cols]))
        out_v[pl.ds(g * 16, 16)] = 1.0 / (1.0 + jnp.exp(-acc))
        return carry

    lax.fori_loop(0, BPW // 16, group, 0)
    pltpu.sync_copy(out_v, out_hbm.at[wid])


@functools.partial(jax.jit, static_argnums=())
def _mf(uidx, iidx, user_factors, item_factors):
    mesh = plsc.VectorSubcoreMesh(core_axis_name="c", subcore_axis_name="s")
    f = functools.partial(
        pl.kernel,
        mesh=mesh,
        compiler_params=pltpu.CompilerParams(
            needs_layout_passes=False, use_tc_tiling_on_sc=False),
        out_type=jax.ShapeDtypeStruct((NW, BPW), jnp.float32),
        scratch_types=[
            pltpu.VMEM((NCHUNK, CHUNK), jnp.int32),
            pltpu.VMEM((NCHUNK, CHUNK), jnp.int32),
            pltpu.VMEM((BPW, N_FACTORS), jnp.float32),
            pltpu.VMEM((BPW, N_FACTORS), jnp.float32),
            pltpu.VMEM((BPW,), jnp.float32),
            pltpu.SemaphoreType.DMA,
        ],
    )(_mf_body)
    return f(uidx, iidx, user_factors, item_factors)


def kernel(X, user_factors, item_factors):
    Xi = X.astype(jnp.int32)
    uidx = Xi[:, 0].reshape(NW, NCHUNK, CHUNK)
    iidx = Xi[:, 1].reshape(NW, NCHUNK, CHUNK)
    out = _mf(uidx, iidx, user_factors, item_factors)
    return out.reshape(BATCH, 1)


# speedup vs baseline: 1.4910x; 1.4910x over previous
import functools

import jax
import jax.numpy as jnp
from jax import lax
from jax.experimental import pallas as pl
from jax.experimental.pallas import tpu as pltpu
from jax.experimental.pallas import tpu_sc as plsc

N_FACTORS = 32
BATCH = 16384
NC = 2
NS = 16
NW = NC * NS
BPW = BATCH // NW
CH = 256
NCH = BPW // CH


def _mf_body(idx_hbm, uf_hbm, if_hbm, out_hbm,
             idx_v, u_v, i_v, out_v, gsem):
    wid = lax.axis_index("s") * NC + lax.axis_index("c")
    pltpu.sync_copy(idx_hbm.at[wid], idx_v)

    def chunk(c, carry):
        def fetch(g, carry2):
            vu = idx_v[0, pl.ds(c * CH + g * 16, 16)]
            vi = idx_v[1, pl.ds(c * CH + g * 16, 16)]
            for k in range(16):
                j = g * 16 + k
                pltpu.async_copy(uf_hbm.at[pl.ds(vu[k], 1)],
                                 u_v.at[pl.ds(j, 1)], gsem)
                pltpu.async_copy(if_hbm.at[pl.ds(vi[k], 1)],
                                 i_v.at[pl.ds(j, 1)], gsem)
            return carry2

        lax.fori_loop(0, CH // 16, fetch, 0)

        def drain(j, carry2):
            pltpu.make_async_copy(
                uf_hbm.at[pl.ds(0, 1)], u_v.at[pl.ds(0, 1)], gsem).wait()
            pltpu.make_async_copy(
                uf_hbm.at[pl.ds(0, 1)], i_v.at[pl.ds(0, 1)], gsem).wait()
            return carry2

        lax.fori_loop(0, CH, drain, 0)

        def group(g, carry2):
            rows = g * 16 + lax.broadcasted_iota(jnp.int32, (16,), 0)
            acc = jnp.zeros((16,), jnp.float32)
            for d in range(N_FACTORS):
                cols = jnp.full((16,), d, jnp.int32)
                acc = acc + (plsc.load_gather(u_v, [rows, cols]) *
                             plsc.load_gather(i_v, [rows, cols]))
            out_v[pl.ds(c * CH + g * 16, 16)] = 1.0 / (1.0 + jnp.exp(-acc))
            return carry2

        lax.fori_loop(0, CH // 16, group, 0)
        return carry

    lax.fori_loop(0, NCH, chunk, 0)
    pltpu.sync_copy(out_v, out_hbm.at[wid])


@jax.jit
def _mf(idx, user_factors, item_factors):
    mesh = plsc.VectorSubcoreMesh(core_axis_name="c", subcore_axis_name="s")
    f = functools.partial(
        pl.kernel,
        mesh=mesh,
        compiler_params=pltpu.CompilerParams(needs_layout_passes=False),
        out_type=jax.ShapeDtypeStruct((NW, BPW), jnp.float32),
        scratch_types=[
            pltpu.VMEM((2, BPW), jnp.int32),
            pltpu.VMEM((CH, N_FACTORS), jnp.float32),
            pltpu.VMEM((CH, N_FACTORS), jnp.float32),
            pltpu.VMEM((BPW,), jnp.float32),
            pltpu.SemaphoreType.DMA,
        ],
    )(_mf_body)
    return f(idx, user_factors, item_factors)


def kernel(X, user_factors, item_factors):
    Xi = X.astype(jnp.int32)
    idx = Xi.reshape(NW, BPW, 2).transpose(0, 2, 1)
    out = _mf(idx, user_factors, item_factors)
    return out.reshape(BATCH, 1)
